# baseline (speedup 1.0000x reference)
import jax
import jax.numpy as jnp
from jax.experimental import pallas as pl


def _add_kernel(p_ref, t_ref, o_ref):
    o_ref[...] = p_ref[...] + t_ref[...]


def _half(patches, pos_table, start, nb):
    B, N, D = patches.shape
    BB = 4
    return pl.pallas_call(
        _add_kernel,
        grid=(nb // BB,),
        in_specs=[
            pl.BlockSpec((BB, N, D), lambda b: (b + start // BB, 0, 0)),
            pl.BlockSpec((N, D), lambda b: (0, 0)),
        ],
        out_specs=pl.BlockSpec((BB, N, D), lambda b: (b, 0, 0)),
        out_shape=jax.ShapeDtypeStruct((nb, N, D), patches.dtype),
    )(patches, pos_table)


def kernel(patches, pos_table):
    B = patches.shape[0]
    h = B // 2
    lo = _half(patches, pos_table, 0, h)
    hi = _half(patches, pos_table, h, B - h)
    return jnp.concatenate([lo, hi], axis=0)


# speedup vs baseline: 2.0114x; 2.0114x over previous
import jax
import jax.numpy as jnp
from jax.experimental import pallas as pl


def _add_kernel(p_ref, t_ref, o_ref):
    o_ref[...] = p_ref[...] + t_ref[...]


def kernel(patches, pos_table):
    B, N, D = patches.shape
    BB = 4
    return pl.pallas_call(
        _add_kernel,
        grid=(B // BB,),
        in_specs=[
            pl.BlockSpec((BB, N, D), lambda b: (b, 0, 0)),
            pl.BlockSpec((N, D), lambda b: (0, 0)),
        ],
        out_specs=pl.BlockSpec((BB, N, D), lambda b: (b, 0, 0)),
        out_shape=jax.ShapeDtypeStruct((B, N, D), patches.dtype),
    )(patches, pos_table)
